# baseline (speedup 1.0000x reference)
import jax
import jax.numpy as jnp
from jax import lax
from jax.experimental import pallas as pl
from jax.experimental.pallas import tpu as pltpu
from jax.experimental.pallas import tpu_sc as plsc

_N = 10000
_E = 320000
_IN = 128
_H1 = 128
_R = 8
_NEG = 0.2

_NC = 2
_NS = 16
_NW = _NC * _NS
_C = 128
_EP = 327680
_CPW = _EP // (_NW * _C)
_EPW = _CPW * _C
_CPW0 = 160
_CPW1 = 0
_NPAD = 10240
_RPT = _NPAD // _NS


def _wmix_body(att_ref, basis_ref, w2_ref):
    w2_ref[...] = jnp.dot(att_ref[...], basis_ref[...],
                          preferred_element_type=jnp.float32)


def _xw_body(x_ref, w_ref, q_ref, k_ref, xw_ref, xq_ref, xk_ref):
    x = x_ref[...]
    qrow = q_ref[...]
    krow = k_ref[...]
    qcols = []
    kcols = []
    for r in range(_R):
        xwr = jnp.dot(x, w_ref[r], preferred_element_type=jnp.float32)
        xw_ref[r] = xwr
        qcols.append(jnp.sum(xwr * qrow, axis=1, keepdims=True))
        kcols.append(jnp.sum(xwr * krow, axis=1, keepdims=True))
    xq_ref[...] = jnp.concatenate(qcols, axis=1)
    xk_ref[...] = jnp.concatenate(kcols, axis=1)


def _eidx_body(src_ref, dst_ref, et_ref, sidx_ref, qidx_ref):
    et = et_ref[...]
    sidx_ref[...] = et * _N + src_ref[...]
    qidx_ref[...] = et * _N + dst_ref[...]


def _sc_attn_body(xwf, xqf, xkf, sidx_hbm, qidx_hbm, dst_hbm,
                  vec_out, den_out,
                  sidx4, qidx4, dst4, qv2, kv2, ex_v, rows2, den_stage,
                  vecacc_sh, den_sh,
                  semq0, semq1, semk0, semk1, semr0, semr1,
                  semi0, semi1, semi2, semi3):
    c = lax.axis_index("c")
    s = lax.axis_index("s")
    row0 = s * _RPT
    wbase = jnp.where(c == 0, s * _CPW0, _NS * _CPW0 + s * _CPW1)
    ncpw = jnp.where(c == 0, _CPW0, _CPW1)
    nquad = jnp.where(c == 0, _CPW0 // 4, _CPW1 // 4)
    semq = (semq0, semq1)
    semk = (semk0, semk1)
    semr = (semr0, semr1)
    semi = (semi0, semi1, semi2, semi3)

    def _zrow(i, carry):
        for j in range(8):
            rows2[0, i, pl.ds(16 * j, 16)] = jnp.zeros((16,), jnp.float32)
        return carry
    lax.fori_loop(0, _C, _zrow, 0)
    for j in range(8):
        ex_v[pl.ds(16 * j, 16)] = jnp.zeros((16,), jnp.float32)
    for b in range(_RPT // _C):
        pltpu.sync_copy(rows2.at[0], vecacc_sh.at[pl.ds(row0 + b * _C, _C)])
        pltpu.sync_copy(ex_v, den_sh.at[pl.ds(row0 + b * _C, _C)])
    plsc.subcore_barrier()

    def _load_idx(chunk, sl):
        base = (wbase + chunk) * _C
        pltpu.async_copy(sidx_hbm.at[pl.ds(base, _C)], sidx4.at[sl],
                         semi[sl])
        pltpu.async_copy(qidx_hbm.at[pl.ds(base, _C)], qidx4.at[sl],
                         semi[sl])
        pltpu.async_copy(dst_hbm.at[pl.ds(base, _C)], dst4.at[sl], semi[sl])

    def _wait_idx(sl):
        pltpu.make_async_copy(sidx_hbm.at[pl.ds(0, _C)], sidx4.at[sl],
                              semi[sl]).wait()
        pltpu.make_async_copy(qidx_hbm.at[pl.ds(0, _C)], qidx4.at[sl],
                              semi[sl]).wait()
        pltpu.make_async_copy(dst_hbm.at[pl.ds(0, _C)], dst4.at[sl],
                              semi[sl]).wait()

    def _issue(sl, b):
        pltpu.async_copy(xqf.at[qidx4.at[sl]], qv2.at[b], semq[b])
        pltpu.async_copy(xkf.at[sidx4.at[sl]], kv2.at[b], semk[b])
        pltpu.async_copy(xwf.at[sidx4.at[sl]], rows2.at[b], semr[b])

    def _wait(b):
        pltpu.make_async_copy(xqf.at[pl.ds(0, _C)], qv2.at[b], semq[b]).wait()
        pltpu.make_async_copy(xkf.at[pl.ds(0, _C)], kv2.at[b], semk[b]).wait()
        pltpu.make_async_copy(xwf.at[pl.ds(0, _C)], rows2.at[b],
                              semr[b]).wait()

    for b in range(2):
        @pl.when(b < ncpw)
        def _():
            _load_idx(b, b)
            _wait_idx(b)
            _issue(b, b)

    def _quad(h, carry):
        for s4 in range(4):
            chunk = 4 * h + s4
            b = s4 % 2
            _wait(b)

            @pl.when(chunk + 2 < ncpw)
            def _():
                _load_idx(chunk + 2, (s4 + 2) % 4)

            for j in range(8):
                a = qv2[b, pl.ds(16 * j, 16)] + kv2[b, pl.ds(16 * j, 16)]
                a = jnp.maximum(a, _NEG * a)
                ex_v[pl.ds(16 * j, 16)] = jnp.exp(a)
            pltpu.sync_copy(ex_v, den_sh.at[dst4.at[s4]], add=True)

            rows_b = rows2.at[b]

            def _scale(g2, carry2):
                ev = ex_v[pl.ds(g2 * 16, 16)]
                for l in range(16):
                    e = g2 * 16 + l
                    sc = ev[l]
                    for j in range(8):
                        rows_b[e, pl.ds(16 * j, 16)] = (
                            rows_b[e, pl.ds(16 * j, 16)] * sc)
                return carry2
            lax.fori_loop(0, _C // 16, _scale, 0)
            pltpu.sync_copy(rows_b, vecacc_sh.at[dst4.at[s4]], add=True)

            @pl.when(chunk + 2 < ncpw)
            def _():
                _wait_idx((s4 + 2) % 4)
                _issue((s4 + 2) % 4, b)
        return carry

    lax.fori_loop(0, nquad, _quad, 0)
    plsc.subcore_barrier()

    for b in range(_RPT // _C):
        off = row0 + b * _C
        pltpu.sync_copy(vecacc_sh.at[pl.ds(off, _C)], rows2.at[0])
        pltpu.sync_copy(rows2.at[0], vec_out.at[c, pl.ds(off, _C)])
    pltpu.sync_copy(den_sh.at[pl.ds(row0, _RPT)], den_stage)
    pltpu.sync_copy(den_stage, den_out.at[c, pl.ds(row0, _RPT)])


def _sc_agg_body(x1f, src_hbm, dst_hbm, agg_out,
                 src4, dst4, rows2, agg_sh,
                 semr0, semr1, semi0, semi1, semi2, semi3):
    c = lax.axis_index("c")
    s = lax.axis_index("s")
    row0 = s * _RPT
    wbase = jnp.where(c == 0, s * _CPW0, _NS * _CPW0 + s * _CPW1)
    ncpw = jnp.where(c == 0, _CPW0, _CPW1)
    nquad = jnp.where(c == 0, _CPW0 // 4, _CPW1 // 4)
    semr = (semr0, semr1)
    semi = (semi0, semi1, semi2, semi3)

    def _zrow(i, carry):
        for j in range(8):
            rows2[0, i, pl.ds(16 * j, 16)] = jnp.zeros((16,), jnp.float32)
        return carry
    lax.fori_loop(0, _C, _zrow, 0)
    for b in range(_RPT // _C):
        pltpu.sync_copy(rows2.at[0], agg_sh.at[pl.ds(row0 + b * _C, _C)])
    plsc.subcore_barrier()

    def _load_idx(chunk, sl):
        base = (wbase + chunk) * _C
        pltpu.async_copy(src_hbm.at[pl.ds(base, _C)], src4.at[sl], semi[sl])
        pltpu.async_copy(dst_hbm.at[pl.ds(base, _C)], dst4.at[sl], semi[sl])

    def _wait_idx(sl):
        pltpu.make_async_copy(src_hbm.at[pl.ds(0, _C)], src4.at[sl],
                              semi[sl]).wait()
        pltpu.make_async_copy(dst_hbm.at[pl.ds(0, _C)], dst4.at[sl],
                              semi[sl]).wait()

    for b in range(2):
        @pl.when(b < ncpw)
        def _():
            _load_idx(b, b)
            _wait_idx(b)
            pltpu.async_copy(x1f.at[src4.at[b]], rows2.at[b], semr[b])

    def _quad(h, carry):
        for s4 in range(4):
            chunk = 4 * h + s4
            b = s4 % 2
            pltpu.make_async_copy(x1f.at[pl.ds(0, _C)], rows2.at[b],
                                  semr[b]).wait()

            @pl.when(chunk + 2 < ncpw)
            def _():
                _load_idx(chunk + 2, (s4 + 2) % 4)

            pltpu.sync_copy(rows2.at[b], agg_sh.at[dst4.at[s4]], add=True)

            @pl.when(chunk + 2 < ncpw)
            def _():
                _wait_idx((s4 + 2) % 4)
                pltpu.async_copy(x1f.at[src4.at[(s4 + 2) % 4]], rows2.at[b],
                                 semr[b])
        return carry

    lax.fori_loop(0, nquad, _quad, 0)
    plsc.subcore_barrier()
    for b in range(_RPT // _C):
        off = row0 + b * _C
        pltpu.sync_copy(agg_sh.at[pl.ds(off, _C)], rows2.at[0])
        pltpu.sync_copy(rows2.at[0], agg_out.at[c, pl.ds(off, _C)])


def _x1_body(vec_ref, den_ref, bias_ref, x1_ref):
    v = vec_ref[0] + vec_ref[1]
    d = den_ref[0] + den_ref[1]
    x1_ref[...] = v / (d[:, None] + 1e-16) + bias_ref[...]


def _out_body(agg_ref, x1_ref, wrelT_ref, wrootT_ref, brel_ref, out_ref):
    agg = agg_ref[0] + agg_ref[1]
    out_ref[...] = (jnp.dot(agg, wrelT_ref[...],
                            preferred_element_type=jnp.float32)
                    + jnp.dot(x1_ref[...], wrootT_ref[...],
                              preferred_element_type=jnp.float32)
                    + brel_ref[...])


def kernel(node_features, edge_index, edge_type, basis, att, q, k, bias1,
           w_rel, b_rel, w_root):
    nb = basis.shape[0]
    src = edge_index[0]
    dst = edge_index[1]

    w2 = pl.pallas_call(
        _wmix_body,
        out_shape=jax.ShapeDtypeStruct((_R, _IN * _H1), jnp.float32),
        in_specs=[pl.BlockSpec((_R, nb), lambda: (0, 0)),
                  pl.BlockSpec((nb, _IN * _H1), lambda: (0, 0))],
        out_specs=pl.BlockSpec((_R, _IN * _H1), lambda: (0, 0)),
    )(att, basis.reshape(nb, _IN * _H1))
    w3 = w2.reshape(_R, _IN, _H1)

    bn = 1000
    grid_n = _N // bn
    xw, xq, xk = pl.pallas_call(
        _xw_body,
        grid=(grid_n,),
        out_shape=[jax.ShapeDtypeStruct((_R, _N, _H1), jnp.float32),
                   jax.ShapeDtypeStruct((_N, _R), jnp.float32),
                   jax.ShapeDtypeStruct((_N, _R), jnp.float32)],
        in_specs=[pl.BlockSpec((bn, _IN), lambda i: (i, 0)),
                  pl.BlockSpec((_R, _IN, _H1), lambda i: (0, 0, 0)),
                  pl.BlockSpec((1, _H1), lambda i: (0, 0)),
                  pl.BlockSpec((1, _H1), lambda i: (0, 0))],
        out_specs=[pl.BlockSpec((_R, bn, _H1), lambda i: (0, i, 0)),
                   pl.BlockSpec((bn, _R), lambda i: (i, 0)),
                   pl.BlockSpec((bn, _R), lambda i: (i, 0))],
    )(node_features, w3, q.reshape(1, _H1), k.reshape(1, _H1))
    xwf = xw.reshape(_R * _N, _H1)
    xqf = xq.T.reshape(_R * _N)
    xkf = xk.T.reshape(_R * _N)

    pad = _EP - _E
    src_p = jnp.concatenate([src, jnp.zeros((pad,), jnp.int32)])
    pad_dst = _N + (jnp.arange(pad, dtype=jnp.int32) % (_NPAD - _N))
    dst_p = jnp.concatenate([dst, pad_dst])
    et_p = jnp.concatenate([edge_type, jnp.zeros((pad,), jnp.int32)])
    epr = _EP // 128

    sidx, qidx = pl.pallas_call(
        _eidx_body,
        out_shape=[jax.ShapeDtypeStruct((epr, 128), jnp.int32),
                   jax.ShapeDtypeStruct((epr, 128), jnp.int32)],
        in_specs=[pl.BlockSpec((epr, 128), lambda: (0, 0))] * 3,
        out_specs=[pl.BlockSpec((epr, 128), lambda: (0, 0))] * 2,
    )(src_p.reshape(epr, 128), dst_p.reshape(epr, 128),
      et_p.reshape(epr, 128))
    sidx = sidx.reshape(_EP)
    qidx = qidx.reshape(_EP)

    mesh = plsc.VectorSubcoreMesh(core_axis_name="c", subcore_axis_name="s")
    vec_part, den_part = pl.kernel(
        _sc_attn_body,
        out_type=[jax.ShapeDtypeStruct((_NC, _NPAD, _H1), jnp.float32),
                  jax.ShapeDtypeStruct((_NC, _NPAD), jnp.float32)],
        mesh=mesh,
        scratch_types=[
            pltpu.VMEM((4, _C), jnp.int32),
            pltpu.VMEM((4, _C), jnp.int32),
            pltpu.VMEM((4, _C), jnp.int32),
            pltpu.VMEM((2, _C), jnp.float32),
            pltpu.VMEM((2, _C), jnp.float32),
            pltpu.VMEM((_C,), jnp.float32),
            pltpu.VMEM((2, _C, _H1), jnp.float32),
            pltpu.VMEM((_RPT,), jnp.float32),
            pltpu.VMEM_SHARED((_NPAD, _H1), jnp.float32),
            pltpu.VMEM_SHARED((_NPAD,), jnp.float32),
            pltpu.SemaphoreType.DMA,
            pltpu.SemaphoreType.DMA,
            pltpu.SemaphoreType.DMA,
            pltpu.SemaphoreType.DMA,
            pltpu.SemaphoreType.DMA,
            pltpu.SemaphoreType.DMA,
            pltpu.SemaphoreType.DMA,
            pltpu.SemaphoreType.DMA,
            pltpu.SemaphoreType.DMA,
            pltpu.SemaphoreType.DMA,
        ],
    )(xwf, xqf, xkf, sidx, qidx, dst_p)

    bn2 = 1024
    grid2 = _NPAD // bn2
    x1 = pl.pallas_call(
        _x1_body,
        grid=(grid2,),
        out_shape=jax.ShapeDtypeStruct((_N, _H1), jnp.float32),
        in_specs=[pl.BlockSpec((_NC, bn2, _H1), lambda i: (0, i, 0)),
                  pl.BlockSpec((_NC, bn2), lambda i: (0, i)),
                  pl.BlockSpec((1, _H1), lambda i: (0, 0))],
        out_specs=pl.BlockSpec((bn2, _H1), lambda i: (i, 0)),
    )(vec_part, den_part, bias1.reshape(1, _H1))

    agg_part = pl.kernel(
        _sc_agg_body,
        out_type=jax.ShapeDtypeStruct((_NC, _NPAD, _H1), jnp.float32),
        mesh=mesh,
        scratch_types=[
            pltpu.VMEM((4, _C), jnp.int32),
            pltpu.VMEM((4, _C), jnp.int32),
            pltpu.VMEM((2, _C, _H1), jnp.float32),
            pltpu.VMEM_SHARED((_NPAD, _H1), jnp.float32),
            pltpu.SemaphoreType.DMA,
            pltpu.SemaphoreType.DMA,
            pltpu.SemaphoreType.DMA,
            pltpu.SemaphoreType.DMA,
            pltpu.SemaphoreType.DMA,
            pltpu.SemaphoreType.DMA,
        ],
    )(x1, src_p, dst_p)

    out = pl.pallas_call(
        _out_body,
        grid=(grid2,),
        out_shape=jax.ShapeDtypeStruct((_N, _H1), jnp.float32),
        in_specs=[pl.BlockSpec((_NC, bn2, _H1), lambda i: (0, i, 0)),
                  pl.BlockSpec((bn2, _H1), lambda i: (i, 0)),
                  pl.BlockSpec((_H1, _H1), lambda i: (0, 0)),
                  pl.BlockSpec((_H1, _H1), lambda i: (0, 0)),
                  pl.BlockSpec((1, _H1), lambda i: (0, 0))],
        out_specs=pl.BlockSpec((bn2, _H1), lambda i: (i, 0)),
    )(agg_part, x1, w_rel.T, w_root.T, b_rel.reshape(1, _H1))
    return out


# speedup vs baseline: 1.1641x; 1.1641x over previous
import jax
import jax.numpy as jnp
from jax import lax
from jax.experimental import pallas as pl
from jax.experimental.pallas import tpu as pltpu
from jax.experimental.pallas import tpu_sc as plsc

_N = 10000
_E = 320000
_IN = 128
_H1 = 128
_R = 8
_NEG = 0.2

_NC = 2
_NS = 16
_NW = _NC * _NS
_C = 128
_EP = 327680
_CPW = _EP // (_NW * _C)
_EPW = _CPW * _C
_CPW0 = 128
_CPW1 = 32
_NPAD = 10240
_RPT = _NPAD // _NS


def _wmix_body(att_ref, basis_ref, w2_ref):
    w2_ref[...] = jnp.dot(att_ref[...], basis_ref[...],
                          preferred_element_type=jnp.float32)


def _xw_body(x_ref, w_ref, q_ref, k_ref, xw_ref, xq_ref, xk_ref):
    x = x_ref[...]
    qrow = q_ref[...]
    krow = k_ref[...]
    qcols = []
    kcols = []
    for r in range(_R):
        xwr = jnp.dot(x, w_ref[r], preferred_element_type=jnp.float32)
        xw_ref[r] = xwr
        qcols.append(jnp.sum(xwr * qrow, axis=1, keepdims=True))
        kcols.append(jnp.sum(xwr * krow, axis=1, keepdims=True))
    xq_ref[...] = jnp.concatenate(qcols, axis=1)
    xk_ref[...] = jnp.concatenate(kcols, axis=1)


def _eidx_body(src_ref, dst_ref, et_ref, sidx_ref, qidx_ref):
    et = et_ref[...]
    sidx_ref[...] = et * _N + src_ref[...]
    qidx_ref[...] = et * _N + dst_ref[...]


def _sc_attn_body(xwf, xqf, xkf, sidx_hbm, qidx_hbm, dst_hbm,
                  vec_out, den_out,
                  sidx4, qidx4, dst4, qv2, kv2, ex_v, rows2, den_stage,
                  vecacc_sh, den_sh,
                  semq0, semq1, semk0, semk1, semr0, semr1,
                  semi0, semi1, semi2, semi3):
    c = lax.axis_index("c")
    s = lax.axis_index("s")
    row0 = s * _RPT
    wbase = jnp.where(c == 0, s * _CPW0, _NS * _CPW0 + s * _CPW1)
    ncpw = jnp.where(c == 0, _CPW0, _CPW1)
    nquad = jnp.where(c == 0, _CPW0 // 4, _CPW1 // 4)
    semq = (semq0, semq1)
    semk = (semk0, semk1)
    semr = (semr0, semr1)
    semi = (semi0, semi1, semi2, semi3)

    def _zrow(i, carry):
        for j in range(8):
            rows2[0, i, pl.ds(16 * j, 16)] = jnp.zeros((16,), jnp.float32)
        return carry
    lax.fori_loop(0, _C, _zrow, 0)
    for j in range(8):
        ex_v[pl.ds(16 * j, 16)] = jnp.zeros((16,), jnp.float32)
    for b in range(_RPT // _C):
        pltpu.sync_copy(rows2.at[0], vecacc_sh.at[pl.ds(row0 + b * _C, _C)])
        pltpu.sync_copy(ex_v, den_sh.at[pl.ds(row0 + b * _C, _C)])
    plsc.subcore_barrier()

    def _load_idx(chunk, sl):
        base = (wbase + chunk) * _C
        pltpu.async_copy(sidx_hbm.at[pl.ds(base, _C)], sidx4.at[sl],
                         semi[sl])
        pltpu.async_copy(qidx_hbm.at[pl.ds(base, _C)], qidx4.at[sl],
                         semi[sl])
        pltpu.async_copy(dst_hbm.at[pl.ds(base, _C)], dst4.at[sl], semi[sl])

    def _wait_idx(sl):
        pltpu.make_async_copy(sidx_hbm.at[pl.ds(0, _C)], sidx4.at[sl],
                              semi[sl]).wait()
        pltpu.make_async_copy(qidx_hbm.at[pl.ds(0, _C)], qidx4.at[sl],
                              semi[sl]).wait()
        pltpu.make_async_copy(dst_hbm.at[pl.ds(0, _C)], dst4.at[sl],
                              semi[sl]).wait()

    def _issue(sl, b):
        pltpu.async_copy(xqf.at[qidx4.at[sl]], qv2.at[b], semq[b])
        pltpu.async_copy(xkf.at[sidx4.at[sl]], kv2.at[b], semk[b])
        pltpu.async_copy(xwf.at[sidx4.at[sl]], rows2.at[b], semr[b])

    def _wait(b):
        pltpu.make_async_copy(xqf.at[pl.ds(0, _C)], qv2.at[b], semq[b]).wait()
        pltpu.make_async_copy(xkf.at[pl.ds(0, _C)], kv2.at[b], semk[b]).wait()
        pltpu.make_async_copy(xwf.at[pl.ds(0, _C)], rows2.at[b],
                              semr[b]).wait()

    with jax.named_scope("prime"):
        for b in range(2):
            @pl.when(b < ncpw)
            def _():
                _load_idx(b, b)
                _wait_idx(b)
                _issue(b, b)

    def _quad(h, carry):
        for s4 in range(4):
            chunk = 4 * h + s4
            b = s4 % 2
            _wait(b)

            @pl.when(chunk + 2 < ncpw)
            def _():
                _load_idx(chunk + 2, (s4 + 2) % 4)

            for j in range(8):
                a = qv2[b, pl.ds(16 * j, 16)] + kv2[b, pl.ds(16 * j, 16)]
                a = jnp.maximum(a, _NEG * a)
                ex_v[pl.ds(16 * j, 16)] = jnp.exp(a)
            pltpu.sync_copy(ex_v, den_sh.at[dst4.at[s4]], add=True)

            rows_b = rows2.at[b]

            def _scale(g2, carry2):
                ev = ex_v[pl.ds(g2 * 16, 16)]
                for l in range(16):
                    e = g2 * 16 + l
                    sc = ev[l]
                    for j in range(8):
                        rows_b[e, pl.ds(16 * j, 16)] = (
                            rows_b[e, pl.ds(16 * j, 16)] * sc)
                return carry2
            lax.fori_loop(0, _C // 16, _scale, 0)
            pltpu.sync_copy(rows_b, vecacc_sh.at[dst4.at[s4]], add=True)

            @pl.when(chunk + 2 < ncpw)
            def _():
                _wait_idx((s4 + 2) % 4)
                _issue((s4 + 2) % 4, b)
        return carry

    with jax.named_scope("mainloop"):
        lax.fori_loop(0, nquad, _quad, 0)
    with jax.named_scope("barrier2"):
        plsc.subcore_barrier()

    for b in range(_RPT // _C):
        off = row0 + b * _C
        pltpu.sync_copy(vecacc_sh.at[pl.ds(off, _C)], rows2.at[0])
        pltpu.sync_copy(rows2.at[0], vec_out.at[c, pl.ds(off, _C)])
    pltpu.sync_copy(den_sh.at[pl.ds(row0, _RPT)], den_stage)
    pltpu.sync_copy(den_stage, den_out.at[c, pl.ds(row0, _RPT)])


def _sc_agg_body(x1f, src_hbm, dst_hbm, agg_out,
                 src4, dst4, rows2, agg_sh,
                 semr0, semr1, semi0, semi1, semi2, semi3):
    c = lax.axis_index("c")
    s = lax.axis_index("s")
    row0 = s * _RPT
    wbase = jnp.where(c == 0, s * _CPW0, _NS * _CPW0 + s * _CPW1)
    ncpw = jnp.where(c == 0, _CPW0, _CPW1)
    nquad = jnp.where(c == 0, _CPW0 // 4, _CPW1 // 4)
    semr = (semr0, semr1)
    semi = (semi0, semi1, semi2, semi3)

    def _zrow(i, carry):
        for j in range(8):
            rows2[0, i, pl.ds(16 * j, 16)] = jnp.zeros((16,), jnp.float32)
        return carry
    lax.fori_loop(0, _C, _zrow, 0)
    for b in range(_RPT // _C):
        pltpu.sync_copy(rows2.at[0], agg_sh.at[pl.ds(row0 + b * _C, _C)])
    plsc.subcore_barrier()

    def _load_idx(chunk, sl):
        base = (wbase + chunk) * _C
        pltpu.async_copy(src_hbm.at[pl.ds(base, _C)], src4.at[sl], semi[sl])
        pltpu.async_copy(dst_hbm.at[pl.ds(base, _C)], dst4.at[sl], semi[sl])

    def _wait_idx(sl):
        pltpu.make_async_copy(src_hbm.at[pl.ds(0, _C)], src4.at[sl],
                              semi[sl]).wait()
        pltpu.make_async_copy(dst_hbm.at[pl.ds(0, _C)], dst4.at[sl],
                              semi[sl]).wait()

    for b in range(2):
        @pl.when(b < ncpw)
        def _():
            _load_idx(b, b)
            _wait_idx(b)
            pltpu.async_copy(x1f.at[src4.at[b]], rows2.at[b], semr[b])

    def _quad(h, carry):
        for s4 in range(4):
            chunk = 4 * h + s4
            b = s4 % 2
            pltpu.make_async_copy(x1f.at[pl.ds(0, _C)], rows2.at[b],
                                  semr[b]).wait()

            @pl.when(chunk + 2 < ncpw)
            def _():
                _load_idx(chunk + 2, (s4 + 2) % 4)

            pltpu.sync_copy(rows2.at[b], agg_sh.at[dst4.at[s4]], add=True)

            @pl.when(chunk + 2 < ncpw)
            def _():
                _wait_idx((s4 + 2) % 4)
                pltpu.async_copy(x1f.at[src4.at[(s4 + 2) % 4]], rows2.at[b],
                                 semr[b])
        return carry

    lax.fori_loop(0, nquad, _quad, 0)
    plsc.subcore_barrier()
    for b in range(_RPT // _C):
        off = row0 + b * _C
        pltpu.sync_copy(agg_sh.at[pl.ds(off, _C)], rows2.at[0])
        pltpu.sync_copy(rows2.at[0], agg_out.at[c, pl.ds(off, _C)])


def _x1_body(vec_ref, den_ref, bias_ref, x1_ref):
    v = vec_ref[0] + vec_ref[1]
    d = den_ref[0] + den_ref[1]
    x1_ref[...] = v / (d[:, None] + 1e-16) + bias_ref[...]


def _out_body(agg_ref, x1_ref, wrelT_ref, wrootT_ref, brel_ref, out_ref):
    agg = agg_ref[0] + agg_ref[1]
    out_ref[...] = (jnp.dot(agg, wrelT_ref[...],
                            preferred_element_type=jnp.float32)
                    + jnp.dot(x1_ref[...], wrootT_ref[...],
                              preferred_element_type=jnp.float32)
                    + brel_ref[...])


def kernel(node_features, edge_index, edge_type, basis, att, q, k, bias1,
           w_rel, b_rel, w_root):
    nb = basis.shape[0]
    src = edge_index[0]
    dst = edge_index[1]

    w2 = pl.pallas_call(
        _wmix_body,
        out_shape=jax.ShapeDtypeStruct((_R, _IN * _H1), jnp.float32),
        in_specs=[pl.BlockSpec((_R, nb), lambda: (0, 0)),
                  pl.BlockSpec((nb, _IN * _H1), lambda: (0, 0))],
        out_specs=pl.BlockSpec((_R, _IN * _H1), lambda: (0, 0)),
    )(att, basis.reshape(nb, _IN * _H1))
    w3 = w2.reshape(_R, _IN, _H1)

    bn = 1000
    grid_n = _N // bn
    xw, xq, xk = pl.pallas_call(
        _xw_body,
        grid=(grid_n,),
        out_shape=[jax.ShapeDtypeStruct((_R, _N, _H1), jnp.float32),
                   jax.ShapeDtypeStruct((_N, _R), jnp.float32),
                   jax.ShapeDtypeStruct((_N, _R), jnp.float32)],
        in_specs=[pl.BlockSpec((bn, _IN), lambda i: (i, 0)),
                  pl.BlockSpec((_R, _IN, _H1), lambda i: (0, 0, 0)),
                  pl.BlockSpec((1, _H1), lambda i: (0, 0)),
                  pl.BlockSpec((1, _H1), lambda i: (0, 0))],
        out_specs=[pl.BlockSpec((_R, bn, _H1), lambda i: (0, i, 0)),
                   pl.BlockSpec((bn, _R), lambda i: (i, 0)),
                   pl.BlockSpec((bn, _R), lambda i: (i, 0))],
    )(node_features, w3, q.reshape(1, _H1), k.reshape(1, _H1))
    xwf = xw.reshape(_R * _N, _H1)
    xqf = xq.T.reshape(_R * _N)
    xkf = xk.T.reshape(_R * _N)

    pad = _EP - _E
    src_p = jnp.concatenate([src, jnp.zeros((pad,), jnp.int32)])
    pad_dst = _N + (jnp.arange(pad, dtype=jnp.int32) % (_NPAD - _N))
    dst_p = jnp.concatenate([dst, pad_dst])
    et_p = jnp.concatenate([edge_type, jnp.zeros((pad,), jnp.int32)])
    epr = _EP // 128

    sidx, qidx = pl.pallas_call(
        _eidx_body,
        out_shape=[jax.ShapeDtypeStruct((epr, 128), jnp.int32),
                   jax.ShapeDtypeStruct((epr, 128), jnp.int32)],
        in_specs=[pl.BlockSpec((epr, 128), lambda: (0, 0))] * 3,
        out_specs=[pl.BlockSpec((epr, 128), lambda: (0, 0))] * 2,
    )(src_p.reshape(epr, 128), dst_p.reshape(epr, 128),
      et_p.reshape(epr, 128))
    sidx = sidx.reshape(_EP)
    qidx = qidx.reshape(_EP)

    mesh = plsc.VectorSubcoreMesh(core_axis_name="c", subcore_axis_name="s")
    vec_part, den_part = pl.kernel(
        _sc_attn_body,
        out_type=[jax.ShapeDtypeStruct((_NC, _NPAD, _H1), jnp.float32),
                  jax.ShapeDtypeStruct((_NC, _NPAD), jnp.float32)],
        mesh=mesh,
        scratch_types=[
            pltpu.VMEM((4, _C), jnp.int32),
            pltpu.VMEM((4, _C), jnp.int32),
            pltpu.VMEM((4, _C), jnp.int32),
            pltpu.VMEM((2, _C), jnp.float32),
            pltpu.VMEM((2, _C), jnp.float32),
            pltpu.VMEM((_C,), jnp.float32),
            pltpu.VMEM((2, _C, _H1), jnp.float32),
            pltpu.VMEM((_RPT,), jnp.float32),
            pltpu.VMEM_SHARED((_NPAD, _H1), jnp.float32),
            pltpu.VMEM_SHARED((_NPAD,), jnp.float32),
            pltpu.SemaphoreType.DMA,
            pltpu.SemaphoreType.DMA,
            pltpu.SemaphoreType.DMA,
            pltpu.SemaphoreType.DMA,
            pltpu.SemaphoreType.DMA,
            pltpu.SemaphoreType.DMA,
            pltpu.SemaphoreType.DMA,
            pltpu.SemaphoreType.DMA,
            pltpu.SemaphoreType.DMA,
            pltpu.SemaphoreType.DMA,
        ],
    )(xwf, xqf, xkf, sidx, qidx, dst_p)

    bn2 = 1024
    grid2 = _NPAD // bn2
    x1 = pl.pallas_call(
        _x1_body,
        grid=(grid2,),
        out_shape=jax.ShapeDtypeStruct((_N, _H1), jnp.float32),
        in_specs=[pl.BlockSpec((_NC, bn2, _H1), lambda i: (0, i, 0)),
                  pl.BlockSpec((_NC, bn2), lambda i: (0, i)),
                  pl.BlockSpec((1, _H1), lambda i: (0, 0))],
        out_specs=pl.BlockSpec((bn2, _H1), lambda i: (i, 0)),
    )(vec_part, den_part, bias1.reshape(1, _H1))

    agg_part = pl.kernel(
        _sc_agg_body,
        out_type=jax.ShapeDtypeStruct((_NC, _NPAD, _H1), jnp.float32),
        mesh=mesh,
        scratch_types=[
            pltpu.VMEM((4, _C), jnp.int32),
            pltpu.VMEM((4, _C), jnp.int32),
            pltpu.VMEM((2, _C, _H1), jnp.float32),
            pltpu.VMEM_SHARED((_NPAD, _H1), jnp.float32),
            pltpu.SemaphoreType.DMA,
            pltpu.SemaphoreType.DMA,
            pltpu.SemaphoreType.DMA,
            pltpu.SemaphoreType.DMA,
            pltpu.SemaphoreType.DMA,
            pltpu.SemaphoreType.DMA,
        ],
    )(x1, src_p, dst_p)

    out = pl.pallas_call(
        _out_body,
        grid=(grid2,),
        out_shape=jax.ShapeDtypeStruct((_N, _H1), jnp.float32),
        in_specs=[pl.BlockSpec((_NC, bn2, _H1), lambda i: (0, i, 0)),
                  pl.BlockSpec((bn2, _H1), lambda i: (i, 0)),
                  pl.BlockSpec((_H1, _H1), lambda i: (0, 0)),
                  pl.BlockSpec((_H1, _H1), lambda i: (0, 0)),
                  pl.BlockSpec((1, _H1), lambda i: (0, 0))],
        out_specs=pl.BlockSpec((bn2, _H1), lambda i: (i, 0)),
    )(agg_part, x1, w_rel.T, w_root.T, b_rel.reshape(1, _H1))
    return out


# speedup vs baseline: 3.5418x; 3.0424x over previous
import jax
import jax.numpy as jnp
from jax import lax
from jax.experimental import pallas as pl
from jax.experimental.pallas import tpu as pltpu
from jax.experimental.pallas import tpu_sc as plsc

_N = 10000
_E = 320000
_IN = 128
_H1 = 128
_R = 8
_NEG = 0.2

_NC = 2
_NS = 16
_NW = _NC * _NS
_C = 128
_EP = 327680
_CPW = _EP // (_NW * _C)
_EPW = _CPW * _C
_CPW0 = 80
_CPW1 = 80
_NPAD = 10240
_RPT = _NPAD // _NS


def _wmix_body(att_ref, basis_ref, w2_ref):
    w2_ref[...] = jnp.dot(att_ref[...], basis_ref[...],
                          preferred_element_type=jnp.float32)


def _xw_body(x_ref, w_ref, q_ref, k_ref, xw_ref, xq_ref, xk_ref):
    x = x_ref[...]
    qrow = q_ref[...]
    krow = k_ref[...]
    qcols = []
    kcols = []
    for r in range(_R):
        xwr = jnp.dot(x, w_ref[r], preferred_element_type=jnp.float32)
        xw_ref[r] = xwr
        qcols.append(jnp.sum(xwr * qrow, axis=1, keepdims=True))
        kcols.append(jnp.sum(xwr * krow, axis=1, keepdims=True))
    xq_ref[...] = jnp.concatenate(qcols, axis=1)
    xk_ref[...] = jnp.concatenate(kcols, axis=1)


def _eidx_body(src_ref, dst_ref, et_ref, sidx_ref, qidx_ref):
    et = et_ref[...]
    sidx_ref[...] = et * _N + src_ref[...]
    qidx_ref[...] = et * _N + dst_ref[...]


def _sc_attn_body(xwf, xqf, xkf, sidx_hbm, qidx_hbm, dst_hbm,
                  vec_out, den_out,
                  sidx4, qidx4, dst4, qv2, kv2, ex_v, rows2, den_stage,
                  vecacc_sh, den_sh,
                  semq0, semq1, semk0, semk1, semr0, semr1,
                  semi0, semi1, semi2, semi3):
    c = lax.axis_index("c")
    s = lax.axis_index("s")
    row0 = s * _RPT
    wbase = jnp.where(c == 0, s * _CPW0, _NS * _CPW0 + s * _CPW1)
    ncpw = jnp.where(c == 0, _CPW0, _CPW1)
    nquad = jnp.where(c == 0, _CPW0 // 4, _CPW1 // 4)
    semq = (semq0, semq1)
    semk = (semk0, semk1)
    semr = (semr0, semr1)
    semi = (semi0, semi1, semi2, semi3)

    def _zrow(i, carry):
        for j in range(8):
            rows2[0, i, pl.ds(16 * j, 16)] = jnp.zeros((16,), jnp.float32)
        return carry
    lax.fori_loop(0, _C, _zrow, 0)
    for j in range(8):
        ex_v[pl.ds(16 * j, 16)] = jnp.zeros((16,), jnp.float32)
    for b in range(_RPT // _C):
        pltpu.sync_copy(rows2.at[0], vecacc_sh.at[pl.ds(row0 + b * _C, _C)])
        pltpu.sync_copy(ex_v, den_sh.at[pl.ds(row0 + b * _C, _C)])
    plsc.subcore_barrier()

    def _load_idx(chunk, sl):
        base = (wbase + chunk) * _C
        pltpu.async_copy(sidx_hbm.at[pl.ds(base, _C)], sidx4.at[sl],
                         semi[sl])
        pltpu.async_copy(qidx_hbm.at[pl.ds(base, _C)], qidx4.at[sl],
                         semi[sl])
        pltpu.async_copy(dst_hbm.at[pl.ds(base, _C)], dst4.at[sl], semi[sl])

    def _wait_idx(sl):
        pltpu.make_async_copy(sidx_hbm.at[pl.ds(0, _C)], sidx4.at[sl],
                              semi[sl]).wait()
        pltpu.make_async_copy(qidx_hbm.at[pl.ds(0, _C)], qidx4.at[sl],
                              semi[sl]).wait()
        pltpu.make_async_copy(dst_hbm.at[pl.ds(0, _C)], dst4.at[sl],
                              semi[sl]).wait()

    def _issue(sl, b):
        pltpu.async_copy(xqf.at[qidx4.at[sl]], qv2.at[b], semq[b])
        pltpu.async_copy(xkf.at[sidx4.at[sl]], kv2.at[b], semk[b])
        pltpu.async_copy(xwf.at[sidx4.at[sl]], rows2.at[b], semr[b])

    def _wait(b):
        pltpu.make_async_copy(xqf.at[pl.ds(0, _C)], qv2.at[b], semq[b]).wait()
        pltpu.make_async_copy(xkf.at[pl.ds(0, _C)], kv2.at[b], semk[b]).wait()
        pltpu.make_async_copy(xwf.at[pl.ds(0, _C)], rows2.at[b],
                              semr[b]).wait()

    for b in range(2):
        @pl.when(b < ncpw)
        def _():
            _load_idx(b, b)
            _wait_idx(b)
            _issue(b, b)

    def _quad(h, carry):
        for s4 in range(4):
            chunk = 4 * h + s4
            b = s4 % 2
            _wait(b)

            @pl.when(chunk + 2 < ncpw)
            def _():
                _load_idx(chunk + 2, (s4 + 2) % 4)

            for j in range(8):
                a = qv2[b, pl.ds(16 * j, 16)] + kv2[b, pl.ds(16 * j, 16)]
                a = jnp.maximum(a, _NEG * a)
                ex_v[pl.ds(16 * j, 16)] = jnp.exp(a)
            pltpu.sync_copy(ex_v, den_sh.at[dst4.at[s4]], add=True)

            rows_b = rows2.at[b]

            def _scale(g2, carry2):
                ev = ex_v[pl.ds(g2 * 16, 16)]
                for l in range(16):
                    e = g2 * 16 + l
                    sc = ev[l]
                    for j in range(8):
                        rows_b[e, pl.ds(16 * j, 16)] = (
                            rows_b[e, pl.ds(16 * j, 16)] * sc)
                return carry2
            lax.fori_loop(0, _C // 16, _scale, 0)
            pltpu.sync_copy(rows_b, vecacc_sh.at[dst4.at[s4]], add=True)

            @pl.when(chunk + 2 < ncpw)
            def _():
                _wait_idx((s4 + 2) % 4)
                _issue((s4 + 2) % 4, b)
        return carry

    lax.fori_loop(0, nquad, _quad, 0)
    plsc.subcore_barrier()

    for b in range(_RPT // _C):
        off = row0 + b * _C
        pltpu.sync_copy(vecacc_sh.at[pl.ds(off, _C)], rows2.at[0])
        pltpu.sync_copy(rows2.at[0], vec_out.at[c, pl.ds(off, _C)])
    pltpu.sync_copy(den_sh.at[pl.ds(row0, _RPT)], den_stage)
    pltpu.sync_copy(den_stage, den_out.at[c, pl.ds(row0, _RPT)])


def _sc_agg_body(x1f, src_hbm, dst_hbm, agg_out,
                 src4, dst4, rows2, agg_sh,
                 semr0, semr1, semi0, semi1, semi2, semi3):
    c = lax.axis_index("c")
    s = lax.axis_index("s")
    row0 = s * _RPT
    wbase = jnp.where(c == 0, s * _CPW0, _NS * _CPW0 + s * _CPW1)
    ncpw = jnp.where(c == 0, _CPW0, _CPW1)
    nquad = jnp.where(c == 0, _CPW0 // 4, _CPW1 // 4)
    semr = (semr0, semr1)
    semi = (semi0, semi1, semi2, semi3)

    def _zrow(i, carry):
        for j in range(8):
            rows2[0, i, pl.ds(16 * j, 16)] = jnp.zeros((16,), jnp.float32)
        return carry
    lax.fori_loop(0, _C, _zrow, 0)
    for b in range(_RPT // _C):
        pltpu.sync_copy(rows2.at[0], agg_sh.at[pl.ds(row0 + b * _C, _C)])
    plsc.subcore_barrier()

    def _load_idx(chunk, sl):
        base = (wbase + chunk) * _C
        pltpu.async_copy(src_hbm.at[pl.ds(base, _C)], src4.at[sl], semi[sl])
        pltpu.async_copy(dst_hbm.at[pl.ds(base, _C)], dst4.at[sl], semi[sl])

    def _wait_idx(sl):
        pltpu.make_async_copy(src_hbm.at[pl.ds(0, _C)], src4.at[sl],
                              semi[sl]).wait()
        pltpu.make_async_copy(dst_hbm.at[pl.ds(0, _C)], dst4.at[sl],
                              semi[sl]).wait()

    for b in range(2):
        @pl.when(b < ncpw)
        def _():
            _load_idx(b, b)
            _wait_idx(b)
            pltpu.async_copy(x1f.at[src4.at[b]], rows2.at[b], semr[b])

    def _quad(h, carry):
        for s4 in range(4):
            chunk = 4 * h + s4
            b = s4 % 2
            pltpu.make_async_copy(x1f.at[pl.ds(0, _C)], rows2.at[b],
                                  semr[b]).wait()

            @pl.when(chunk + 2 < ncpw)
            def _():
                _load_idx(chunk + 2, (s4 + 2) % 4)

            pltpu.sync_copy(rows2.at[b], agg_sh.at[dst4.at[s4]], add=True)

            @pl.when(chunk + 2 < ncpw)
            def _():
                _wait_idx((s4 + 2) % 4)
                pltpu.async_copy(x1f.at[src4.at[(s4 + 2) % 4]], rows2.at[b],
                                 semr[b])
        return carry

    lax.fori_loop(0, nquad, _quad, 0)
    plsc.subcore_barrier()
    for b in range(_RPT // _C):
        off = row0 + b * _C
        pltpu.sync_copy(agg_sh.at[pl.ds(off, _C)], rows2.at[0])
        pltpu.sync_copy(rows2.at[0], agg_out.at[c, pl.ds(off, _C)])


def _x1_body(vec_ref, den_ref, bias_ref, x1_ref):
    v = vec_ref[0] + vec_ref[1]
    d = den_ref[0] + den_ref[1]
    x1_ref[...] = v / (d[:, None] + 1e-16) + bias_ref[...]


def _out_body(agg_ref, x1_ref, wrelT_ref, wrootT_ref, brel_ref, out_ref):
    agg = agg_ref[0] + agg_ref[1]
    out_ref[...] = (jnp.dot(agg, wrelT_ref[...],
                            preferred_element_type=jnp.float32)
                    + jnp.dot(x1_ref[...], wrootT_ref[...],
                              preferred_element_type=jnp.float32)
                    + brel_ref[...])


def kernel(node_features, edge_index, edge_type, basis, att, q, k, bias1,
           w_rel, b_rel, w_root):
    nb = basis.shape[0]
    src = edge_index[0]
    dst = edge_index[1]

    w2 = pl.pallas_call(
        _wmix_body,
        out_shape=jax.ShapeDtypeStruct((_R, _IN * _H1), jnp.float32),
        in_specs=[pl.BlockSpec((_R, nb), lambda: (0, 0)),
                  pl.BlockSpec((nb, _IN * _H1), lambda: (0, 0))],
        out_specs=pl.BlockSpec((_R, _IN * _H1), lambda: (0, 0)),
    )(att, basis.reshape(nb, _IN * _H1))
    w3 = w2.reshape(_R, _IN, _H1)

    bn = 1000
    grid_n = _N // bn
    xw, xq, xk = pl.pallas_call(
        _xw_body,
        grid=(grid_n,),
        out_shape=[jax.ShapeDtypeStruct((_R, _N, _H1), jnp.float32),
                   jax.ShapeDtypeStruct((_N, _R), jnp.float32),
                   jax.ShapeDtypeStruct((_N, _R), jnp.float32)],
        in_specs=[pl.BlockSpec((bn, _IN), lambda i: (i, 0)),
                  pl.BlockSpec((_R, _IN, _H1), lambda i: (0, 0, 0)),
                  pl.BlockSpec((1, _H1), lambda i: (0, 0)),
                  pl.BlockSpec((1, _H1), lambda i: (0, 0))],
        out_specs=[pl.BlockSpec((_R, bn, _H1), lambda i: (0, i, 0)),
                   pl.BlockSpec((bn, _R), lambda i: (i, 0)),
                   pl.BlockSpec((bn, _R), lambda i: (i, 0))],
    )(node_features, w3, q.reshape(1, _H1), k.reshape(1, _H1))
    xwf = xw.reshape(_R * _N, _H1)
    xqf = xq.T.reshape(_R * _N)
    xkf = xk.T.reshape(_R * _N)

    pad = _EP - _E
    pad_src = jnp.arange(pad, dtype=jnp.int32) % _N
    src_p = jnp.concatenate([src, pad_src])
    pad_dst = _N + (jnp.arange(pad, dtype=jnp.int32) % (_NPAD - _N))
    dst_p = jnp.concatenate([dst, pad_dst])
    et_p = jnp.concatenate([edge_type, jnp.zeros((pad,), jnp.int32)])
    epr = _EP // 128

    sidx, qidx = pl.pallas_call(
        _eidx_body,
        out_shape=[jax.ShapeDtypeStruct((epr, 128), jnp.int32),
                   jax.ShapeDtypeStruct((epr, 128), jnp.int32)],
        in_specs=[pl.BlockSpec((epr, 128), lambda: (0, 0))] * 3,
        out_specs=[pl.BlockSpec((epr, 128), lambda: (0, 0))] * 2,
    )(src_p.reshape(epr, 128), dst_p.reshape(epr, 128),
      et_p.reshape(epr, 128))
    sidx = sidx.reshape(_EP)
    qidx = qidx.reshape(_EP)

    mesh = plsc.VectorSubcoreMesh(core_axis_name="c", subcore_axis_name="s")
    vec_part, den_part = pl.kernel(
        _sc_attn_body,
        out_type=[jax.ShapeDtypeStruct((_NC, _NPAD, _H1), jnp.float32),
                  jax.ShapeDtypeStruct((_NC, _NPAD), jnp.float32)],
        mesh=mesh,
        scratch_types=[
            pltpu.VMEM((4, _C), jnp.int32),
            pltpu.VMEM((4, _C), jnp.int32),
            pltpu.VMEM((4, _C), jnp.int32),
            pltpu.VMEM((2, _C), jnp.float32),
            pltpu.VMEM((2, _C), jnp.float32),
            pltpu.VMEM((_C,), jnp.float32),
            pltpu.VMEM((2, _C, _H1), jnp.float32),
            pltpu.VMEM((_RPT,), jnp.float32),
            pltpu.VMEM_SHARED((_NPAD, _H1), jnp.float32),
            pltpu.VMEM_SHARED((_NPAD,), jnp.float32),
            pltpu.SemaphoreType.DMA,
            pltpu.SemaphoreType.DMA,
            pltpu.SemaphoreType.DMA,
            pltpu.SemaphoreType.DMA,
            pltpu.SemaphoreType.DMA,
            pltpu.SemaphoreType.DMA,
            pltpu.SemaphoreType.DMA,
            pltpu.SemaphoreType.DMA,
            pltpu.SemaphoreType.DMA,
            pltpu.SemaphoreType.DMA,
        ],
    )(xwf, xqf, xkf, sidx, qidx, dst_p)

    bn2 = 1024
    grid2 = _NPAD // bn2
    x1 = pl.pallas_call(
        _x1_body,
        grid=(grid2,),
        out_shape=jax.ShapeDtypeStruct((_N, _H1), jnp.float32),
        in_specs=[pl.BlockSpec((_NC, bn2, _H1), lambda i: (0, i, 0)),
                  pl.BlockSpec((_NC, bn2), lambda i: (0, i)),
                  pl.BlockSpec((1, _H1), lambda i: (0, 0))],
        out_specs=pl.BlockSpec((bn2, _H1), lambda i: (i, 0)),
    )(vec_part, den_part, bias1.reshape(1, _H1))

    agg_part = pl.kernel(
        _sc_agg_body,
        out_type=jax.ShapeDtypeStruct((_NC, _NPAD, _H1), jnp.float32),
        mesh=mesh,
        scratch_types=[
            pltpu.VMEM((4, _C), jnp.int32),
            pltpu.VMEM((4, _C), jnp.int32),
            pltpu.VMEM((2, _C, _H1), jnp.float32),
            pltpu.VMEM_SHARED((_NPAD, _H1), jnp.float32),
            pltpu.SemaphoreType.DMA,
            pltpu.SemaphoreType.DMA,
            pltpu.SemaphoreType.DMA,
            pltpu.SemaphoreType.DMA,
            pltpu.SemaphoreType.DMA,
            pltpu.SemaphoreType.DMA,
        ],
    )(x1, src_p, dst_p)

    out = pl.pallas_call(
        _out_body,
        grid=(grid2,),
        out_shape=jax.ShapeDtypeStruct((_N, _H1), jnp.float32),
        in_specs=[pl.BlockSpec((_NC, bn2, _H1), lambda i: (0, i, 0)),
                  pl.BlockSpec((bn2, _H1), lambda i: (i, 0)),
                  pl.BlockSpec((_H1, _H1), lambda i: (0, 0)),
                  pl.BlockSpec((_H1, _H1), lambda i: (0, 0)),
                  pl.BlockSpec((1, _H1), lambda i: (0, 0))],
        out_specs=pl.BlockSpec((bn2, _H1), lambda i: (i, 0)),
    )(agg_part, x1, w_rel.T, w_root.T, b_rel.reshape(1, _H1))
    return out


# speedup vs baseline: 3.5527x; 1.0031x over previous
import jax
import jax.numpy as jnp
from jax import lax
from jax.experimental import pallas as pl
from jax.experimental.pallas import tpu as pltpu
from jax.experimental.pallas import tpu_sc as plsc

_N = 10000
_E = 320000
_IN = 128
_H1 = 128
_R = 8
_NEG = 0.2

_NC = 2
_NS = 16
_NW = _NC * _NS
_C = 128
_EP = 327680
_CPW = _EP // (_NW * _C)
_EPW = _CPW * _C
_CPW0 = 80
_CPW1 = 80
_NPAD = 10240
_RPT = _NPAD // _NS


def _wmix_body(att_ref, basis_ref, w2_ref):
    w2_ref[...] = jnp.dot(att_ref[...], basis_ref[...],
                          preferred_element_type=jnp.float32)


def _xw_body(x_ref, w_ref, q_ref, k_ref, xw_ref, xq_ref, xk_ref):
    x = x_ref[...]
    qrow = q_ref[...]
    krow = k_ref[...]
    qcols = []
    kcols = []
    for r in range(_R):
        xwr = jnp.dot(x, w_ref[r], preferred_element_type=jnp.float32)
        xw_ref[r] = xwr
        qcols.append(jnp.sum(xwr * qrow, axis=1, keepdims=True))
        kcols.append(jnp.sum(xwr * krow, axis=1, keepdims=True))
    xq_ref[...] = jnp.concatenate(qcols, axis=1)
    xk_ref[...] = jnp.concatenate(kcols, axis=1)


def _eidx_body(src_ref, dst_ref, et_ref, sidx_ref, qidx_ref):
    et = et_ref[...]
    sidx_ref[...] = et * _N + src_ref[...]
    qidx_ref[...] = et * _N + dst_ref[...]


def _sc_attn_body(xwf, xqf, xkf, sidx_hbm, qidx_hbm, dst_hbm,
                  vec_out, den_out,
                  sidx4, qidx4, dst4, qv2, kv2, ex_v, rows2, den_stage,
                  vecacc_sh, den_sh,
                  semq0, semq1, semk0, semk1, semr0, semr1,
                  semi0, semi1, semi2, semi3):
    c = lax.axis_index("c")
    s = lax.axis_index("s")
    row0 = s * _RPT
    wbase = jnp.where(c == 0, s * _CPW0, _NS * _CPW0 + s * _CPW1)
    ncpw = jnp.where(c == 0, _CPW0, _CPW1)
    nquad = jnp.where(c == 0, _CPW0 // 4, _CPW1 // 4)
    semq = (semq0, semq1)
    semk = (semk0, semk1)
    semr = (semr0, semr1)
    semi = (semi0, semi1, semi2, semi3)

    def _zrow(i, carry):
        for j in range(8):
            rows2[0, i, pl.ds(16 * j, 16)] = jnp.zeros((16,), jnp.float32)
        return carry
    lax.fori_loop(0, _C, _zrow, 0)
    for j in range(8):
        ex_v[pl.ds(16 * j, 16)] = jnp.zeros((16,), jnp.float32)
    for b in range(_RPT // _C):
        pltpu.sync_copy(rows2.at[0], vecacc_sh.at[pl.ds(row0 + b * _C, _C)])
        pltpu.sync_copy(ex_v, den_sh.at[pl.ds(row0 + b * _C, _C)])
    plsc.subcore_barrier()

    def _load_idx(chunk, sl):
        base = (wbase + chunk) * _C
        pltpu.async_copy(sidx_hbm.at[pl.ds(base, _C)], sidx4.at[sl],
                         semi[sl])
        pltpu.async_copy(qidx_hbm.at[pl.ds(base, _C)], qidx4.at[sl],
                         semi[sl])
        pltpu.async_copy(dst_hbm.at[pl.ds(base, _C)], dst4.at[sl], semi[sl])

    def _wait_idx(sl):
        pltpu.make_async_copy(sidx_hbm.at[pl.ds(0, _C)], sidx4.at[sl],
                              semi[sl]).wait()
        pltpu.make_async_copy(qidx_hbm.at[pl.ds(0, _C)], qidx4.at[sl],
                              semi[sl]).wait()
        pltpu.make_async_copy(dst_hbm.at[pl.ds(0, _C)], dst4.at[sl],
                              semi[sl]).wait()

    def _issue(sl, b):
        pltpu.async_copy(xqf.at[qidx4.at[sl]], qv2.at[b], semq[b])
        pltpu.async_copy(xkf.at[sidx4.at[sl]], kv2.at[b], semk[b])
        pltpu.async_copy(xwf.at[sidx4.at[sl]], rows2.at[b], semr[b])

    def _wait(b):
        pltpu.make_async_copy(xqf.at[pl.ds(0, _C)], qv2.at[b], semq[b]).wait()
        pltpu.make_async_copy(xkf.at[pl.ds(0, _C)], kv2.at[b], semk[b]).wait()
        pltpu.make_async_copy(xwf.at[pl.ds(0, _C)], rows2.at[b],
                              semr[b]).wait()

    for b in range(2):
        @pl.when(b < ncpw)
        def _():
            _load_idx(b, b)
            _wait_idx(b)
            _issue(b, b)

    def _quad(h, carry):
        for s4 in range(4):
            chunk = 4 * h + s4
            b = s4 % 2
            _wait(b)

            @pl.when(chunk + 2 < ncpw)
            def _():
                _load_idx(chunk + 2, (s4 + 2) % 4)

            for j in range(8):
                a = qv2[b, pl.ds(16 * j, 16)] + kv2[b, pl.ds(16 * j, 16)]
                a = jnp.maximum(a, _NEG * a)
                ex_v[pl.ds(16 * j, 16)] = jnp.exp(a)
            pltpu.sync_copy(ex_v, den_sh.at[dst4.at[s4]], add=True)

            rows_b = rows2.at[b]

            @plsc.parallel_loop(0, _C // 16, step=1, unroll=2)
            def _scale(g2):
                ev = ex_v[pl.ds(g2 * 16, 16)]
                for l in range(16):
                    e = g2 * 16 + l
                    sc = ev[l]
                    for j in range(8):
                        rows_b[e, pl.ds(16 * j, 16)] = (
                            rows_b[e, pl.ds(16 * j, 16)] * sc)
            pltpu.sync_copy(rows_b, vecacc_sh.at[dst4.at[s4]], add=True)

            @pl.when(chunk + 2 < ncpw)
            def _():
                _wait_idx((s4 + 2) % 4)
                _issue((s4 + 2) % 4, b)
        return carry

    lax.fori_loop(0, nquad, _quad, 0)
    plsc.subcore_barrier()

    for b in range(_RPT // _C):
        off = row0 + b * _C
        pltpu.sync_copy(vecacc_sh.at[pl.ds(off, _C)], rows2.at[0])
        pltpu.sync_copy(rows2.at[0], vec_out.at[c, pl.ds(off, _C)])
    pltpu.sync_copy(den_sh.at[pl.ds(row0, _RPT)], den_stage)
    pltpu.sync_copy(den_stage, den_out.at[c, pl.ds(row0, _RPT)])


def _sc_agg_body(x1f, src_hbm, dst_hbm, agg_out,
                 src4, dst4, rows2, agg_sh,
                 semr0, semr1, semi0, semi1, semi2, semi3):
    c = lax.axis_index("c")
    s = lax.axis_index("s")
    row0 = s * _RPT
    wbase = jnp.where(c == 0, s * _CPW0, _NS * _CPW0 + s * _CPW1)
    ncpw = jnp.where(c == 0, _CPW0, _CPW1)
    nquad = jnp.where(c == 0, _CPW0 // 4, _CPW1 // 4)
    semr = (semr0, semr1)
    semi = (semi0, semi1, semi2, semi3)

    def _zrow(i, carry):
        for j in range(8):
            rows2[0, i, pl.ds(16 * j, 16)] = jnp.zeros((16,), jnp.float32)
        return carry
    lax.fori_loop(0, _C, _zrow, 0)
    for b in range(_RPT // _C):
        pltpu.sync_copy(rows2.at[0], agg_sh.at[pl.ds(row0 + b * _C, _C)])
    plsc.subcore_barrier()

    def _load_idx(chunk, sl):
        base = (wbase + chunk) * _C
        pltpu.async_copy(src_hbm.at[pl.ds(base, _C)], src4.at[sl], semi[sl])
        pltpu.async_copy(dst_hbm.at[pl.ds(base, _C)], dst4.at[sl], semi[sl])

    def _wait_idx(sl):
        pltpu.make_async_copy(src_hbm.at[pl.ds(0, _C)], src4.at[sl],
                              semi[sl]).wait()
        pltpu.make_async_copy(dst_hbm.at[pl.ds(0, _C)], dst4.at[sl],
                              semi[sl]).wait()

    for b in range(2):
        @pl.when(b < ncpw)
        def _():
            _load_idx(b, b)
            _wait_idx(b)
            pltpu.async_copy(x1f.at[src4.at[b]], rows2.at[b], semr[b])

    def _quad(h, carry):
        for s4 in range(4):
            chunk = 4 * h + s4
            b = s4 % 2
            pltpu.make_async_copy(x1f.at[pl.ds(0, _C)], rows2.at[b],
                                  semr[b]).wait()

            @pl.when(chunk + 2 < ncpw)
            def _():
                _load_idx(chunk + 2, (s4 + 2) % 4)

            pltpu.sync_copy(rows2.at[b], agg_sh.at[dst4.at[s4]], add=True)

            @pl.when(chunk + 2 < ncpw)
            def _():
                _wait_idx((s4 + 2) % 4)
                pltpu.async_copy(x1f.at[src4.at[(s4 + 2) % 4]], rows2.at[b],
                                 semr[b])
        return carry

    lax.fori_loop(0, nquad, _quad, 0)
    plsc.subcore_barrier()
    for b in range(_RPT // _C):
        off = row0 + b * _C
        pltpu.sync_copy(agg_sh.at[pl.ds(off, _C)], rows2.at[0])
        pltpu.sync_copy(rows2.at[0], agg_out.at[c, pl.ds(off, _C)])


def _x1_body(vec_ref, den_ref, bias_ref, x1_ref):
    v = vec_ref[0] + vec_ref[1]
    d = den_ref[0] + den_ref[1]
    x1_ref[...] = v / (d[:, None] + 1e-16) + bias_ref[...]


def _out_body(agg_ref, x1_ref, wrelT_ref, wrootT_ref, brel_ref, out_ref):
    agg = agg_ref[0] + agg_ref[1]
    out_ref[...] = (jnp.dot(agg, wrelT_ref[...],
                            preferred_element_type=jnp.float32)
                    + jnp.dot(x1_ref[...], wrootT_ref[...],
                              preferred_element_type=jnp.float32)
                    + brel_ref[...])


def kernel(node_features, edge_index, edge_type, basis, att, q, k, bias1,
           w_rel, b_rel, w_root):
    nb = basis.shape[0]
    src = edge_index[0]
    dst = edge_index[1]

    w2 = pl.pallas_call(
        _wmix_body,
        out_shape=jax.ShapeDtypeStruct((_R, _IN * _H1), jnp.float32),
        in_specs=[pl.BlockSpec((_R, nb), lambda: (0, 0)),
                  pl.BlockSpec((nb, _IN * _H1), lambda: (0, 0))],
        out_specs=pl.BlockSpec((_R, _IN * _H1), lambda: (0, 0)),
    )(att, basis.reshape(nb, _IN * _H1))
    w3 = w2.reshape(_R, _IN, _H1)

    bn = 1000
    grid_n = _N // bn
    xw, xq, xk = pl.pallas_call(
        _xw_body,
        grid=(grid_n,),
        out_shape=[jax.ShapeDtypeStruct((_R, _N, _H1), jnp.float32),
                   jax.ShapeDtypeStruct((_N, _R), jnp.float32),
                   jax.ShapeDtypeStruct((_N, _R), jnp.float32)],
        in_specs=[pl.BlockSpec((bn, _IN), lambda i: (i, 0)),
                  pl.BlockSpec((_R, _IN, _H1), lambda i: (0, 0, 0)),
                  pl.BlockSpec((1, _H1), lambda i: (0, 0)),
                  pl.BlockSpec((1, _H1), lambda i: (0, 0))],
        out_specs=[pl.BlockSpec((_R, bn, _H1), lambda i: (0, i, 0)),
                   pl.BlockSpec((bn, _R), lambda i: (i, 0)),
                   pl.BlockSpec((bn, _R), lambda i: (i, 0))],
    )(node_features, w3, q.reshape(1, _H1), k.reshape(1, _H1))
    xwf = xw.reshape(_R * _N, _H1)
    xqf = xq.T.reshape(_R * _N)
    xkf = xk.T.reshape(_R * _N)

    pad = _EP - _E
    pad_src = jnp.arange(pad, dtype=jnp.int32) % _N
    src_p = jnp.concatenate([src, pad_src])
    pad_dst = _N + (jnp.arange(pad, dtype=jnp.int32) % (_NPAD - _N))
    dst_p = jnp.concatenate([dst, pad_dst])
    et_p = jnp.concatenate([edge_type, jnp.zeros((pad,), jnp.int32)])
    epr = _EP // 128

    sidx, qidx = pl.pallas_call(
        _eidx_body,
        out_shape=[jax.ShapeDtypeStruct((epr, 128), jnp.int32),
                   jax.ShapeDtypeStruct((epr, 128), jnp.int32)],
        in_specs=[pl.BlockSpec((epr, 128), lambda: (0, 0))] * 3,
        out_specs=[pl.BlockSpec((epr, 128), lambda: (0, 0))] * 2,
    )(src_p.reshape(epr, 128), dst_p.reshape(epr, 128),
      et_p.reshape(epr, 128))
    sidx = sidx.reshape(_EP)
    qidx = qidx.reshape(_EP)

    mesh = plsc.VectorSubcoreMesh(core_axis_name="c", subcore_axis_name="s")
    vec_part, den_part = pl.kernel(
        _sc_attn_body,
        out_type=[jax.ShapeDtypeStruct((_NC, _NPAD, _H1), jnp.float32),
                  jax.ShapeDtypeStruct((_NC, _NPAD), jnp.float32)],
        mesh=mesh,
        scratch_types=[
            pltpu.VMEM((4, _C), jnp.int32),
            pltpu.VMEM((4, _C), jnp.int32),
            pltpu.VMEM((4, _C), jnp.int32),
            pltpu.VMEM((2, _C), jnp.float32),
            pltpu.VMEM((2, _C), jnp.float32),
            pltpu.VMEM((_C,), jnp.float32),
            pltpu.VMEM((2, _C, _H1), jnp.float32),
            pltpu.VMEM((_RPT,), jnp.float32),
            pltpu.VMEM_SHARED((_NPAD, _H1), jnp.float32),
            pltpu.VMEM_SHARED((_NPAD,), jnp.float32),
            pltpu.SemaphoreType.DMA,
            pltpu.SemaphoreType.DMA,
            pltpu.SemaphoreType.DMA,
            pltpu.SemaphoreType.DMA,
            pltpu.SemaphoreType.DMA,
            pltpu.SemaphoreType.DMA,
            pltpu.SemaphoreType.DMA,
            pltpu.SemaphoreType.DMA,
            pltpu.SemaphoreType.DMA,
            pltpu.SemaphoreType.DMA,
        ],
    )(xwf, xqf, xkf, sidx, qidx, dst_p)

    bn2 = 1024
    grid2 = _NPAD // bn2
    x1 = pl.pallas_call(
        _x1_body,
        grid=(grid2,),
        out_shape=jax.ShapeDtypeStruct((_N, _H1), jnp.float32),
        in_specs=[pl.BlockSpec((_NC, bn2, _H1), lambda i: (0, i, 0)),
                  pl.BlockSpec((_NC, bn2), lambda i: (0, i)),
                  pl.BlockSpec((1, _H1), lambda i: (0, 0))],
        out_specs=pl.BlockSpec((bn2, _H1), lambda i: (i, 0)),
    )(vec_part, den_part, bias1.reshape(1, _H1))

    agg_part = pl.kernel(
        _sc_agg_body,
        out_type=jax.ShapeDtypeStruct((_NC, _NPAD, _H1), jnp.float32),
        mesh=mesh,
        scratch_types=[
            pltpu.VMEM((4, _C), jnp.int32),
            pltpu.VMEM((4, _C), jnp.int32),
            pltpu.VMEM((2, _C, _H1), jnp.float32),
            pltpu.VMEM_SHARED((_NPAD, _H1), jnp.float32),
            pltpu.SemaphoreType.DMA,
            pltpu.SemaphoreType.DMA,
            pltpu.SemaphoreType.DMA,
            pltpu.SemaphoreType.DMA,
            pltpu.SemaphoreType.DMA,
            pltpu.SemaphoreType.DMA,
        ],
    )(x1, src_p, dst_p)

    out = pl.pallas_call(
        _out_body,
        grid=(grid2,),
        out_shape=jax.ShapeDtypeStruct((_N, _H1), jnp.float32),
        in_specs=[pl.BlockSpec((_NC, bn2, _H1), lambda i: (0, i, 0)),
                  pl.BlockSpec((bn2, _H1), lambda i: (i, 0)),
                  pl.BlockSpec((_H1, _H1), lambda i: (0, 0)),
                  pl.BlockSpec((_H1, _H1), lambda i: (0, 0)),
                  pl.BlockSpec((1, _H1), lambda i: (0, 0))],
        out_specs=pl.BlockSpec((bn2, _H1), lambda i: (i, 0)),
    )(agg_part, x1, w_rel.T, w_root.T, b_rel.reshape(1, _H1))
    return out


# speedup vs baseline: 3.5936x; 1.0115x over previous
import jax
import jax.numpy as jnp
from jax import lax
from jax.experimental import pallas as pl
from jax.experimental.pallas import tpu as pltpu
from jax.experimental.pallas import tpu_sc as plsc

_N = 10000
_E = 320000
_IN = 128
_H1 = 128
_R = 8
_NEG = 0.2

_NC = 2
_NS = 16
_NW = _NC * _NS
_C = 128
_EP = 327680
_CPW = _EP // (_NW * _C)
_EPW = _CPW * _C
_CPW0 = 80
_CPW1 = 80
_NPAD = 10240
_RPT = _NPAD // _NS


def _wmix_body(att_ref, basis_ref, w2_ref):
    w2_ref[...] = jnp.dot(att_ref[...], basis_ref[...],
                          preferred_element_type=jnp.float32)


def _xw_body(x_ref, w_ref, q_ref, k_ref, xw_ref, xq_ref, xk_ref):
    x = x_ref[...]
    qrow = q_ref[...]
    krow = k_ref[...]
    qcols = []
    kcols = []
    for r in range(_R):
        xwr = jnp.dot(x, w_ref[r], preferred_element_type=jnp.float32)
        xw_ref[r] = xwr
        qcols.append(jnp.sum(xwr * qrow, axis=1, keepdims=True))
        kcols.append(jnp.sum(xwr * krow, axis=1, keepdims=True))
    xq_ref[...] = jnp.concatenate(qcols, axis=1)
    xk_ref[...] = jnp.concatenate(kcols, axis=1)


def _eidx_body(src_ref, dst_ref, et_ref, sidx_ref, qidx_ref):
    et = et_ref[...]
    sidx_ref[...] = et * _N + src_ref[...]
    qidx_ref[...] = et * _N + dst_ref[...]


def _sc_attn_body(xwf, xqf, xkf, sidx_hbm, qidx_hbm, dst_hbm,
                  vec_out, den_out,
                  sidx4, qidx4, dst4, qv2, kv2, ex_v, rows2, den_stage,
                  vecacc_sh, den_sh,
                  semq0, semq1, semk0, semk1, semr0, semr1,
                  semi0, semi1, semi2, semi3, semd):
    c = lax.axis_index("c")
    s = lax.axis_index("s")
    row0 = s * _RPT
    wbase = jnp.where(c == 0, s * _CPW0, _NS * _CPW0 + s * _CPW1)
    ncpw = jnp.where(c == 0, _CPW0, _CPW1)
    nquad = jnp.where(c == 0, _CPW0 // 4, _CPW1 // 4)
    semq = (semq0, semq1)
    semk = (semk0, semk1)
    semr = (semr0, semr1)
    semi = (semi0, semi1, semi2, semi3)

    def _zrow(i, carry):
        for j in range(8):
            rows2[0, i, pl.ds(16 * j, 16)] = jnp.zeros((16,), jnp.float32)
        return carry
    lax.fori_loop(0, _C, _zrow, 0)
    for j in range(8):
        ex_v[pl.ds(16 * j, 16)] = jnp.zeros((16,), jnp.float32)
    for b in range(_RPT // _C):
        pltpu.sync_copy(rows2.at[0], vecacc_sh.at[pl.ds(row0 + b * _C, _C)])
        pltpu.sync_copy(ex_v, den_sh.at[pl.ds(row0 + b * _C, _C)])
    plsc.subcore_barrier()

    def _load_idx(chunk, sl):
        base = (wbase + chunk) * _C
        pltpu.async_copy(sidx_hbm.at[pl.ds(base, _C)], sidx4.at[sl],
                         semi[sl])
        pltpu.async_copy(qidx_hbm.at[pl.ds(base, _C)], qidx4.at[sl],
                         semi[sl])
        pltpu.async_copy(dst_hbm.at[pl.ds(base, _C)], dst4.at[sl], semi[sl])

    def _wait_idx(sl):
        pltpu.make_async_copy(sidx_hbm.at[pl.ds(0, _C)], sidx4.at[sl],
                              semi[sl]).wait()
        pltpu.make_async_copy(qidx_hbm.at[pl.ds(0, _C)], qidx4.at[sl],
                              semi[sl]).wait()
        pltpu.make_async_copy(dst_hbm.at[pl.ds(0, _C)], dst4.at[sl],
                              semi[sl]).wait()

    def _issue(sl, b):
        pltpu.async_copy(xqf.at[qidx4.at[sl]], qv2.at[b], semq[b])
        pltpu.async_copy(xkf.at[sidx4.at[sl]], kv2.at[b], semk[b])
        pltpu.async_copy(xwf.at[sidx4.at[sl]], rows2.at[b], semr[b])

    def _wait(b):
        pltpu.make_async_copy(xqf.at[pl.ds(0, _C)], qv2.at[b], semq[b]).wait()
        pltpu.make_async_copy(xkf.at[pl.ds(0, _C)], kv2.at[b], semk[b]).wait()
        pltpu.make_async_copy(xwf.at[pl.ds(0, _C)], rows2.at[b],
                              semr[b]).wait()

    for b in range(2):
        @pl.when(b < ncpw)
        def _():
            _load_idx(b, b)
            _wait_idx(b)
            _issue(b, b)

    def _quad(h, carry):
        for s4 in range(4):
            chunk = 4 * h + s4
            b = s4 % 2
            _wait(b)

            @pl.when(chunk + 2 < ncpw)
            def _():
                _load_idx(chunk + 2, (s4 + 2) % 4)

            for j in range(8):
                a = qv2[b, pl.ds(16 * j, 16)] + kv2[b, pl.ds(16 * j, 16)]
                a = jnp.maximum(a, _NEG * a)
                ex_v[pl.ds(16 * j, 16)] = jnp.exp(a)
            dcp = pltpu.async_copy(ex_v, den_sh.at[dst4.at[s4]], semd,
                                   add=True)

            rows_b = rows2.at[b]

            @plsc.parallel_loop(0, _C // 16, step=1, unroll=2)
            def _scale(g2):
                ev = ex_v[pl.ds(g2 * 16, 16)]
                for l in range(16):
                    e = g2 * 16 + l
                    sc = ev[l]
                    for j in range(8):
                        rows_b[e, pl.ds(16 * j, 16)] = (
                            rows_b[e, pl.ds(16 * j, 16)] * sc)
            dcp.wait()
            pltpu.sync_copy(rows_b, vecacc_sh.at[dst4.at[s4]], add=True)

            @pl.when(chunk + 2 < ncpw)
            def _():
                _wait_idx((s4 + 2) % 4)
                _issue((s4 + 2) % 4, b)
        return carry

    lax.fori_loop(0, nquad, _quad, 0)
    plsc.subcore_barrier()

    for b in range(_RPT // _C):
        off = row0 + b * _C
        pltpu.sync_copy(vecacc_sh.at[pl.ds(off, _C)], rows2.at[0])
        pltpu.sync_copy(rows2.at[0], vec_out.at[c, pl.ds(off, _C)])
    pltpu.sync_copy(den_sh.at[pl.ds(row0, _RPT)], den_stage)
    pltpu.sync_copy(den_stage, den_out.at[c, pl.ds(row0, _RPT)])


def _sc_agg_body(x1f, src_hbm, dst_hbm, agg_out,
                 src4, dst4, rows2, agg_sh,
                 semr0, semr1, semi0, semi1, semi2, semi3):
    c = lax.axis_index("c")
    s = lax.axis_index("s")
    row0 = s * _RPT
    wbase = jnp.where(c == 0, s * _CPW0, _NS * _CPW0 + s * _CPW1)
    ncpw = jnp.where(c == 0, _CPW0, _CPW1)
    nquad = jnp.where(c == 0, _CPW0 // 4, _CPW1 // 4)
    semr = (semr0, semr1)
    semi = (semi0, semi1, semi2, semi3)

    def _zrow(i, carry):
        for j in range(8):
            rows2[0, i, pl.ds(16 * j, 16)] = jnp.zeros((16,), jnp.float32)
        return carry
    lax.fori_loop(0, _C, _zrow, 0)
    for b in range(_RPT // _C):
        pltpu.sync_copy(rows2.at[0], agg_sh.at[pl.ds(row0 + b * _C, _C)])
    plsc.subcore_barrier()

    def _load_idx(chunk, sl):
        base = (wbase + chunk) * _C
        pltpu.async_copy(src_hbm.at[pl.ds(base, _C)], src4.at[sl], semi[sl])
        pltpu.async_copy(dst_hbm.at[pl.ds(base, _C)], dst4.at[sl], semi[sl])

    def _wait_idx(sl):
        pltpu.make_async_copy(src_hbm.at[pl.ds(0, _C)], src4.at[sl],
                              semi[sl]).wait()
        pltpu.make_async_copy(dst_hbm.at[pl.ds(0, _C)], dst4.at[sl],
                              semi[sl]).wait()

    for b in range(2):
        @pl.when(b < ncpw)
        def _():
            _load_idx(b, b)
            _wait_idx(b)
            pltpu.async_copy(x1f.at[src4.at[b]], rows2.at[b], semr[b])

    def _quad(h, carry):
        for s4 in range(4):
            chunk = 4 * h + s4
            b = s4 % 2
            pltpu.make_async_copy(x1f.at[pl.ds(0, _C)], rows2.at[b],
                                  semr[b]).wait()

            @pl.when(chunk + 2 < ncpw)
            def _():
                _load_idx(chunk + 2, (s4 + 2) % 4)

            pltpu.sync_copy(rows2.at[b], agg_sh.at[dst4.at[s4]], add=True)

            @pl.when(chunk + 2 < ncpw)
            def _():
                _wait_idx((s4 + 2) % 4)
                pltpu.async_copy(x1f.at[src4.at[(s4 + 2) % 4]], rows2.at[b],
                                 semr[b])
        return carry

    lax.fori_loop(0, nquad, _quad, 0)
    plsc.subcore_barrier()
    for b in range(_RPT // _C):
        off = row0 + b * _C
        pltpu.sync_copy(agg_sh.at[pl.ds(off, _C)], rows2.at[0])
        pltpu.sync_copy(rows2.at[0], agg_out.at[c, pl.ds(off, _C)])


def _x1_body(vec_ref, den_ref, bias_ref, x1_ref):
    v = vec_ref[0] + vec_ref[1]
    d = den_ref[0] + den_ref[1]
    x1_ref[...] = v / (d[:, None] + 1e-16) + bias_ref[...]


def _out_body(agg_ref, x1_ref, wrelT_ref, wrootT_ref, brel_ref, out_ref):
    agg = agg_ref[0] + agg_ref[1]
    out_ref[...] = (jnp.dot(agg, wrelT_ref[...],
                            preferred_element_type=jnp.float32)
                    + jnp.dot(x1_ref[...], wrootT_ref[...],
                              preferred_element_type=jnp.float32)
                    + brel_ref[...])


def kernel(node_features, edge_index, edge_type, basis, att, q, k, bias1,
           w_rel, b_rel, w_root):
    nb = basis.shape[0]
    src = edge_index[0]
    dst = edge_index[1]

    w2 = pl.pallas_call(
        _wmix_body,
        out_shape=jax.ShapeDtypeStruct((_R, _IN * _H1), jnp.float32),
        in_specs=[pl.BlockSpec((_R, nb), lambda: (0, 0)),
                  pl.BlockSpec((nb, _IN * _H1), lambda: (0, 0))],
        out_specs=pl.BlockSpec((_R, _IN * _H1), lambda: (0, 0)),
    )(att, basis.reshape(nb, _IN * _H1))
    w3 = w2.reshape(_R, _IN, _H1)

    bn = 1000
    grid_n = _N // bn
    xw, xq, xk = pl.pallas_call(
        _xw_body,
        grid=(grid_n,),
        out_shape=[jax.ShapeDtypeStruct((_R, _N, _H1), jnp.float32),
                   jax.ShapeDtypeStruct((_N, _R), jnp.float32),
                   jax.ShapeDtypeStruct((_N, _R), jnp.float32)],
        in_specs=[pl.BlockSpec((bn, _IN), lambda i: (i, 0)),
                  pl.BlockSpec((_R, _IN, _H1), lambda i: (0, 0, 0)),
                  pl.BlockSpec((1, _H1), lambda i: (0, 0)),
                  pl.BlockSpec((1, _H1), lambda i: (0, 0))],
        out_specs=[pl.BlockSpec((_R, bn, _H1), lambda i: (0, i, 0)),
                   pl.BlockSpec((bn, _R), lambda i: (i, 0)),
                   pl.BlockSpec((bn, _R), lambda i: (i, 0))],
    )(node_features, w3, q.reshape(1, _H1), k.reshape(1, _H1))
    xwf = xw.reshape(_R * _N, _H1)
    xqf = xq.T.reshape(_R * _N)
    xkf = xk.T.reshape(_R * _N)

    pad = _EP - _E
    pad_src = jnp.arange(pad, dtype=jnp.int32) % _N
    src_p = jnp.concatenate([src, pad_src])
    pad_dst = _N + (jnp.arange(pad, dtype=jnp.int32) % (_NPAD - _N))
    dst_p = jnp.concatenate([dst, pad_dst])
    et_p = jnp.concatenate([edge_type, jnp.zeros((pad,), jnp.int32)])
    epr = _EP // 128

    sidx, qidx = pl.pallas_call(
        _eidx_body,
        out_shape=[jax.ShapeDtypeStruct((epr, 128), jnp.int32),
                   jax.ShapeDtypeStruct((epr, 128), jnp.int32)],
        in_specs=[pl.BlockSpec((epr, 128), lambda: (0, 0))] * 3,
        out_specs=[pl.BlockSpec((epr, 128), lambda: (0, 0))] * 2,
    )(src_p.reshape(epr, 128), dst_p.reshape(epr, 128),
      et_p.reshape(epr, 128))
    sidx = sidx.reshape(_EP)
    qidx = qidx.reshape(_EP)

    mesh = plsc.VectorSubcoreMesh(core_axis_name="c", subcore_axis_name="s")
    vec_part, den_part = pl.kernel(
        _sc_attn_body,
        out_type=[jax.ShapeDtypeStruct((_NC, _NPAD, _H1), jnp.float32),
                  jax.ShapeDtypeStruct((_NC, _NPAD), jnp.float32)],
        mesh=mesh,
        scratch_types=[
            pltpu.VMEM((4, _C), jnp.int32),
            pltpu.VMEM((4, _C), jnp.int32),
            pltpu.VMEM((4, _C), jnp.int32),
            pltpu.VMEM((2, _C), jnp.float32),
            pltpu.VMEM((2, _C), jnp.float32),
            pltpu.VMEM((_C,), jnp.float32),
            pltpu.VMEM((2, _C, _H1), jnp.float32),
            pltpu.VMEM((_RPT,), jnp.float32),
            pltpu.VMEM_SHARED((_NPAD, _H1), jnp.float32),
            pltpu.VMEM_SHARED((_NPAD,), jnp.float32),
            pltpu.SemaphoreType.DMA,
            pltpu.SemaphoreType.DMA,
            pltpu.SemaphoreType.DMA,
            pltpu.SemaphoreType.DMA,
            pltpu.SemaphoreType.DMA,
            pltpu.SemaphoreType.DMA,
            pltpu.SemaphoreType.DMA,
            pltpu.SemaphoreType.DMA,
            pltpu.SemaphoreType.DMA,
            pltpu.SemaphoreType.DMA,
            pltpu.SemaphoreType.DMA,
        ],
    )(xwf, xqf, xkf, sidx, qidx, dst_p)

    bn2 = 1024
    grid2 = _NPAD // bn2
    x1 = pl.pallas_call(
        _x1_body,
        grid=(grid2,),
        out_shape=jax.ShapeDtypeStruct((_N, _H1), jnp.float32),
        in_specs=[pl.BlockSpec((_NC, bn2, _H1), lambda i: (0, i, 0)),
                  pl.BlockSpec((_NC, bn2), lambda i: (0, i)),
                  pl.BlockSpec((1, _H1), lambda i: (0, 0))],
        out_specs=pl.BlockSpec((bn2, _H1), lambda i: (i, 0)),
    )(vec_part, den_part, bias1.reshape(1, _H1))

    agg_part = pl.kernel(
        _sc_agg_body,
        out_type=jax.ShapeDtypeStruct((_NC, _NPAD, _H1), jnp.float32),
        mesh=mesh,
        scratch_types=[
            pltpu.VMEM((4, _C), jnp.int32),
            pltpu.VMEM((4, _C), jnp.int32),
            pltpu.VMEM((2, _C, _H1), jnp.float32),
            pltpu.VMEM_SHARED((_NPAD, _H1), jnp.float32),
            pltpu.SemaphoreType.DMA,
            pltpu.SemaphoreType.DMA,
            pltpu.SemaphoreType.DMA,
            pltpu.SemaphoreType.DMA,
            pltpu.SemaphoreType.DMA,
            pltpu.SemaphoreType.DMA,
        ],
    )(x1, src_p, dst_p)

    out = pl.pallas_call(
        _out_body,
        grid=(grid2,),
        out_shape=jax.ShapeDtypeStruct((_N, _H1), jnp.float32),
        in_specs=[pl.BlockSpec((_NC, bn2, _H1), lambda i: (0, i, 0)),
                  pl.BlockSpec((bn2, _H1), lambda i: (i, 0)),
                  pl.BlockSpec((_H1, _H1), lambda i: (0, 0)),
                  pl.BlockSpec((_H1, _H1), lambda i: (0, 0)),
                  pl.BlockSpec((1, _H1), lambda i: (0, 0))],
        out_specs=pl.BlockSpec((bn2, _H1), lambda i: (i, 0)),
    )(agg_part, x1, w_rel.T, w_root.T, b_rel.reshape(1, _H1))
    return out
